# baseline (speedup 1.0000x reference)
import functools

import jax
import jax.numpy as jnp
from jax import lax
from jax.experimental import pallas as pl
from jax.experimental.pallas import tpu as pltpu
from jax.experimental.pallas import tpu_sc as plsc

LAMBDA_C = 1.0
_L = 16
_NACC = 6


def _sc_partials(feat_p, label, centers_p):
    B, D2 = feat_p.shape
    NC, NS = 2, 16
    NW = NC * NS
    RPW = B // NW
    RSUB = 16
    NSUB = RPW // RSUB
    NBUF = 4

    mesh = plsc.VectorSubcoreMesh(core_axis_name="c", subcore_axis_name="s")

    @functools.partial(
        pl.kernel,
        mesh=mesh,
        out_type=jax.ShapeDtypeStruct((NW, _L), jnp.float32),
        scratch_types=[
            pltpu.VMEM((RPW,), jnp.int32),
            pltpu.VMEM((NBUF, RSUB, D2), jnp.int32),
            pltpu.VMEM((NBUF, RSUB, D2), jnp.int32),
            pltpu.VMEM((_L,), jnp.float32),
            pltpu.SemaphoreType.DMA,
            pltpu.SemaphoreType.DMA,
            pltpu.SemaphoreType.DMA,
            pltpu.SemaphoreType.DMA,
            pltpu.SemaphoreType.DMA,
            pltpu.SemaphoreType.DMA,
            pltpu.SemaphoreType.DMA,
            pltpu.SemaphoreType.DMA,
        ],
    )
    def k(feat_hbm, label_hbm, centers_hbm, out_hbm,
          idx_v, feat_v, crows_v, part_v,
          sf0, sf1, sf2, sf3, sc0, sc1, sc2, sc3):
        wid = lax.axis_index("s") * NC + lax.axis_index("c")
        base = wid * RPW
        sems_f = (sf0, sf1, sf2, sf3)
        sems_c = (sc0, sc1, sc2, sc3)
        pltpu.sync_copy(label_hbm.at[pl.ds(base, RPW)], idx_v)

        def issue(s, b):
            row0 = base + s * RSUB
            pltpu.async_copy(feat_hbm.at[pl.ds(row0, RSUB)],
                             feat_v.at[b], sems_f[b])
            pltpu.async_copy(centers_hbm.at[idx_v.at[pl.ds(s * RSUB, RSUB)]],
                             crows_v.at[b], sems_c[b])

        def wait(s, b):
            row0 = base + s * RSUB
            pltpu.make_async_copy(feat_hbm.at[pl.ds(row0, RSUB)],
                                  feat_v.at[b], sems_f[b]).wait()
            pltpu.make_async_copy(
                centers_hbm.at[idx_v.at[pl.ds(s * RSUB, RSUB)]],
                crows_v.at[b], sems_c[b]).wait()

        for b in range(NBUF):
            issue(b, b)

        hi_mask = jnp.full((_L,), -65536, dtype=jnp.int32)

        def halves(u):
            lo = lax.bitcast_convert_type(u << 16, jnp.float32)
            hi = lax.bitcast_convert_type(u & hi_mask, jnp.float32)
            return lo, hi

        def compute_sub(b, accs):
            def row_body(r, accs):
                accs = list(accs)
                for c in range(D2 // _L):
                    uf = feat_v[b, r, pl.ds(c * _L, _L)]
                    ug = crows_v[b, r, pl.ds(c * _L, _L)]
                    f0, f1 = halves(uf)
                    g0, g1 = halves(ug)
                    d0 = f0 - g0
                    d1 = f1 - g1
                    j = (2 * c) % _NACC
                    accs[j] = accs[j] + d0 * d0
                    accs[j + 1] = accs[j + 1] + d1 * d1
                return tuple(accs)
            return lax.fori_loop(0, RSUB, row_body, accs)

        accs = tuple(jnp.zeros((_L,), jnp.float32) for _ in range(_NACC))

        def group_body(g, accs):
            for b in range(NBUF):
                s = g * NBUF + b
                wait(s, b)
                accs = compute_sub(b, accs)

                @pl.when(s + NBUF < NSUB)
                def _():
                    issue(s + NBUF, b)
            return accs

        accs = lax.fori_loop(0, NSUB // NBUF, group_body, accs)

        total = accs[0]
        for j in range(1, _NACC):
            total = total + accs[j]
        part_v[...] = total
        pltpu.sync_copy(part_v, out_hbm.at[wid])

    return k(feat_p, label, centers_p)


def _pack_bf16_words(x):
    b, d = x.shape
    xb = x.astype(jnp.bfloat16).reshape(b, d // 2, 2)
    return lax.bitcast_convert_type(xb, jnp.int32)


def kernel(feat, label, centers):
    B = feat.shape[0]
    parts = _sc_partials(_pack_bf16_words(feat),
                         label.astype(jnp.int32),
                         _pack_bf16_words(centers))
    return LAMBDA_C / 2.0 / B * jnp.sqrt(jnp.sum(parts))


# speedup vs baseline: 2.1626x; 2.1626x over previous
import functools

import jax
import jax.numpy as jnp
from jax import lax
from jax.experimental import pallas as pl
from jax.experimental.pallas import tpu as pltpu
from jax.experimental.pallas import tpu_sc as plsc

LAMBDA_C = 1.0
_L = 16
_NACC = 6


def _sc_partials(feat_p, label, centers_p):
    B, D2 = feat_p.shape
    NC, NS = 2, 16
    NW = NC * NS
    RPW = B // NW
    RSUB = 16
    NSUB = RPW // RSUB
    NBUF = 4

    mesh = plsc.VectorSubcoreMesh(core_axis_name="c", subcore_axis_name="s")

    @functools.partial(
        pl.kernel,
        mesh=mesh,
        out_type=jax.ShapeDtypeStruct((NW, _L), jnp.float32),
        scratch_types=[
            pltpu.VMEM((RPW,), jnp.int32),
            pltpu.VMEM((NBUF, RSUB, D2), jnp.int32),
            pltpu.VMEM((NBUF, RSUB, D2), jnp.int32),
            pltpu.VMEM((_L,), jnp.float32),
            pltpu.SemaphoreType.DMA,
            pltpu.SemaphoreType.DMA,
            pltpu.SemaphoreType.DMA,
            pltpu.SemaphoreType.DMA,
            pltpu.SemaphoreType.DMA,
            pltpu.SemaphoreType.DMA,
            pltpu.SemaphoreType.DMA,
            pltpu.SemaphoreType.DMA,
        ],
    )
    def k(feat_hbm, label_hbm, centers_hbm, out_hbm,
          idx_v, feat_v, crows_v, part_v,
          sf0, sf1, sf2, sf3, sc0, sc1, sc2, sc3):
        wid = lax.axis_index("s") * NC + lax.axis_index("c")
        base = wid * RPW
        sems_f = (sf0, sf1, sf2, sf3)
        sems_c = (sc0, sc1, sc2, sc3)
        pltpu.sync_copy(label_hbm.at[pl.ds(base, RPW)], idx_v)

        def issue(s, b):
            row0 = base + s * RSUB
            pltpu.async_copy(feat_hbm.at[pl.ds(row0, RSUB)],
                             feat_v.at[b], sems_f[b])
            pltpu.async_copy(centers_hbm.at[idx_v.at[pl.ds(s * RSUB, RSUB)]],
                             crows_v.at[b], sems_c[b])

        def wait(s, b):
            row0 = base + s * RSUB
            pltpu.make_async_copy(feat_hbm.at[pl.ds(row0, RSUB)],
                                  feat_v.at[b], sems_f[b]).wait()
            pltpu.make_async_copy(
                centers_hbm.at[idx_v.at[pl.ds(s * RSUB, RSUB)]],
                crows_v.at[b], sems_c[b]).wait()

        for b in range(NBUF):
            issue(b, b)

        hi_mask = jnp.full((_L,), -65536, dtype=jnp.int32)

        def halves(u):
            lo = lax.bitcast_convert_type(u << 16, jnp.float32)
            hi = lax.bitcast_convert_type(u & hi_mask, jnp.float32)
            return lo, hi

        def compute_sub(b, accs):
            def row_body(r, accs):
                accs = list(accs)
                for c in range(D2 // _L):
                    uf = feat_v[b, r, pl.ds(c * _L, _L)]
                    ug = crows_v[b, r, pl.ds(c * _L, _L)]
                    f0, f1 = halves(uf)
                    g0, g1 = halves(ug)
                    d0 = f0 - g0
                    d1 = f1 - g1
                    j = (2 * c) % _NACC
                    accs[j] = accs[j] + d0 * d0
                    accs[j + 1] = accs[j + 1] + d1 * d1
                return tuple(accs)
            return lax.fori_loop(0, RSUB, row_body, accs)

        accs = tuple(jnp.zeros((_L,), jnp.float32) for _ in range(_NACC))

        def group_body(g, accs):
            for b in range(NBUF):
                s = g * NBUF + b
                wait(s, b)
                accs = compute_sub(b, accs)

                @pl.when(s + NBUF < NSUB)
                def _():
                    issue(s + NBUF, b)
            return accs

        accs = lax.fori_loop(0, NSUB // NBUF, group_body, accs)

        total = accs[0]
        for j in range(1, _NACC):
            total = total + accs[j]
        part_v[...] = total
        pltpu.sync_copy(part_v, out_hbm.at[wid])

    return k(feat_p, label, centers_p)


def _pack_halfwords(x):
    d2 = x.shape[1] // 2
    u = lax.bitcast_convert_type(x, jnp.int32)
    lo = lax.shift_right_logical(u[:, :d2], 16)
    hi = u[:, d2:] & jnp.int32(-65536)
    return hi | lo


def kernel(feat, label, centers):
    B = feat.shape[0]
    parts = _sc_partials(_pack_halfwords(feat),
                         label.astype(jnp.int32),
                         _pack_halfwords(centers))
    return LAMBDA_C / 2.0 / B * jnp.sqrt(jnp.sum(parts))


# speedup vs baseline: 2.3940x; 1.1070x over previous
import functools

import jax
import jax.numpy as jnp
from jax import lax
from jax.experimental import pallas as pl
from jax.experimental.pallas import tpu as pltpu
from jax.experimental.pallas import tpu_sc as plsc

LAMBDA_C = 1.0
_L = 16
_NACC = 6


def _sc_partials(feat, label, centers):
    B, D = feat.shape
    NC, NS = 2, 16
    NW = NC * NS
    RPW = B // NW
    RSUB = 16
    NSUB = RPW // RSUB
    NBUF = 4

    mesh = plsc.VectorSubcoreMesh(core_axis_name="c", subcore_axis_name="s")

    @functools.partial(
        pl.kernel,
        mesh=mesh,
        out_type=jax.ShapeDtypeStruct((NW, _L), jnp.float32),
        scratch_types=[
            pltpu.VMEM((RPW,), jnp.int32),
            pltpu.VMEM((NBUF, RSUB, D), jnp.float32),
            pltpu.VMEM((NBUF, RSUB, D), jnp.float32),
            pltpu.VMEM((_L,), jnp.float32),
            pltpu.SemaphoreType.DMA,
            pltpu.SemaphoreType.DMA,
            pltpu.SemaphoreType.DMA,
            pltpu.SemaphoreType.DMA,
            pltpu.SemaphoreType.DMA,
            pltpu.SemaphoreType.DMA,
            pltpu.SemaphoreType.DMA,
            pltpu.SemaphoreType.DMA,
        ],
    )
    def k(feat_hbm, label_hbm, centers_hbm, out_hbm,
          idx_v, feat_v, crows_v, part_v,
          sf0, sf1, sf2, sf3, sc0, sc1, sc2, sc3):
        wid = lax.axis_index("s") * NC + lax.axis_index("c")
        base = wid * RPW
        sems_f = (sf0, sf1, sf2, sf3)
        sems_c = (sc0, sc1, sc2, sc3)
        pltpu.sync_copy(label_hbm.at[pl.ds(base, RPW)], idx_v)

        def issue(s, b):
            row0 = base + s * RSUB
            pltpu.async_copy(feat_hbm.at[pl.ds(row0, RSUB)],
                             feat_v.at[b], sems_f[b])
            pltpu.async_copy(centers_hbm.at[idx_v.at[pl.ds(s * RSUB, RSUB)]],
                             crows_v.at[b], sems_c[b])

        def wait(s, b):
            row0 = base + s * RSUB
            pltpu.make_async_copy(feat_hbm.at[pl.ds(row0, RSUB)],
                                  feat_v.at[b], sems_f[b]).wait()
            pltpu.make_async_copy(
                centers_hbm.at[idx_v.at[pl.ds(s * RSUB, RSUB)]],
                crows_v.at[b], sems_c[b]).wait()

        for b in range(NBUF):
            issue(b, b)

        def compute_sub(b, accs):
            def row_body(r, accs):
                accs = list(accs)
                for c in range(D // _L):
                    f = feat_v[b, r, pl.ds(c * _L, _L)]
                    g = crows_v[b, r, pl.ds(c * _L, _L)]
                    d = f - g
                    j = c % _NACC
                    accs[j] = accs[j] + d * d
                return tuple(accs)
            return lax.fori_loop(0, RSUB, row_body, accs)

        accs = tuple(jnp.zeros((_L,), jnp.float32) for _ in range(_NACC))

        def group_body(g, accs):
            for b in range(NBUF):
                s = g * NBUF + b
                wait(s, b)
                accs = compute_sub(b, accs)

                @pl.when(s + NBUF < NSUB)
                def _():
                    issue(s + NBUF, b)
            return accs

        accs = lax.fori_loop(0, NSUB // NBUF, group_body, accs)

        total = accs[0]
        for j in range(1, _NACC):
            total = total + accs[j]
        part_v[...] = total
        pltpu.sync_copy(part_v, out_hbm.at[wid])

    return k(feat, label, centers)


def kernel(feat, label, centers):
    B = feat.shape[0]
    parts = _sc_partials(feat, label.astype(jnp.int32), centers)
    return LAMBDA_C / 2.0 / B * jnp.sqrt(jnp.sum(parts))


# speedup vs baseline: 2.4036x; 1.0040x over previous
import functools

import jax
import jax.numpy as jnp
from jax import lax
from jax.experimental import pallas as pl
from jax.experimental.pallas import tpu as pltpu
from jax.experimental.pallas import tpu_sc as plsc

LAMBDA_C = 1.0
_L = 16
_NACC = 6


def _sc_partials(feat, label, centers):
    B, D = feat.shape
    NC, NS = 2, 16
    NW = NC * NS
    RPW = B // NW
    RSUB = 16
    NSUB = RPW // RSUB
    NBUF = 4

    mesh = plsc.VectorSubcoreMesh(core_axis_name="c", subcore_axis_name="s")

    @functools.partial(
        pl.kernel,
        mesh=mesh,
        out_type=jax.ShapeDtypeStruct((NW, _L), jnp.float32),
        scratch_types=[
            pltpu.VMEM((RPW,), jnp.int32),
            pltpu.VMEM((NBUF, RSUB, D), jnp.float32),
            pltpu.VMEM((NBUF, RSUB, D), jnp.float32),
            pltpu.VMEM((_L,), jnp.float32),
            pltpu.SemaphoreType.DMA,
            pltpu.SemaphoreType.DMA,
            pltpu.SemaphoreType.DMA,
            pltpu.SemaphoreType.DMA,
            pltpu.SemaphoreType.DMA,
            pltpu.SemaphoreType.DMA,
            pltpu.SemaphoreType.DMA,
            pltpu.SemaphoreType.DMA,
        ],
    )
    def k(feat_hbm, label_hbm, centers_hbm, out_hbm,
          idx_v, feat_v, crows_v, part_v,
          sf0, sf1, sf2, sf3, sc0, sc1, sc2, sc3):
        wid = lax.axis_index("s") * NC + lax.axis_index("c")
        base = wid * RPW
        sems_f = (sf0, sf1, sf2, sf3)
        sems_c = (sc0, sc1, sc2, sc3)
        pltpu.sync_copy(label_hbm.at[pl.ds(base, RPW)], idx_v)

        def issue(s, b):
            row0 = base + s * RSUB
            pltpu.async_copy(centers_hbm.at[idx_v.at[pl.ds(s * RSUB, RSUB)]],
                             crows_v.at[b], sems_c[b])
            pltpu.async_copy(feat_hbm.at[pl.ds(row0, RSUB)],
                             feat_v.at[b], sems_f[b])

        def wait(s, b):
            row0 = base + s * RSUB
            pltpu.make_async_copy(feat_hbm.at[pl.ds(row0, RSUB)],
                                  feat_v.at[b], sems_f[b]).wait()
            pltpu.make_async_copy(
                centers_hbm.at[idx_v.at[pl.ds(s * RSUB, RSUB)]],
                crows_v.at[b], sems_c[b]).wait()

        for b in range(NBUF):
            issue(b, b)

        def compute_sub(b, accs):
            def row_body(r, accs):
                accs = list(accs)
                for c in range(D // _L):
                    f = feat_v[b, r, pl.ds(c * _L, _L)]
                    g = crows_v[b, r, pl.ds(c * _L, _L)]
                    d = f - g
                    j = c % _NACC
                    accs[j] = accs[j] + d * d
                return tuple(accs)
            return lax.fori_loop(0, RSUB, row_body, accs)

        accs = tuple(jnp.zeros((_L,), jnp.float32) for _ in range(_NACC))

        def group_body(g, accs):
            for b in range(NBUF):
                s = g * NBUF + b
                wait(s, b)
                accs = compute_sub(b, accs)

                @pl.when(s + NBUF < NSUB)
                def _():
                    issue(s + NBUF, b)
            return accs

        accs = lax.fori_loop(0, NSUB // NBUF, group_body, accs)

        total = accs[0]
        for j in range(1, _NACC):
            total = total + accs[j]
        part_v[...] = total
        pltpu.sync_copy(part_v, out_hbm.at[wid])

    return k(feat, label, centers)


def kernel(feat, label, centers):
    B = feat.shape[0]
    parts = _sc_partials(feat, label.astype(jnp.int32), centers)
    return LAMBDA_C / 2.0 / B * jnp.sqrt(jnp.sum(parts))
